# baseline (speedup 1.0000x reference)
import functools

import jax
import jax.numpy as jnp
from jax import lax
from jax.experimental import pallas as pl
from jax.experimental.pallas import tpu as pltpu
from jax.experimental.pallas import tpu_sc as plsc

N = 10000
D = 128
E = 320000
EPS = 1e-5
BATCH = 100

NC = 2
NS = 16
NW = NC * NS
EW = E // NW
K = 80
NCH = EW // K
NBUF = 3
NBI = 6
STR = 624
STR_LAST = N - (NS - 1) * STR

_mesh = plsc.VectorSubcoreMesh(core_axis_name="c", subcore_axis_name="s")


@functools.partial(
    pl.kernel,
    out_type=jax.ShapeDtypeStruct((NC, N, D), jnp.float32),
    mesh=_mesh,
    scratch_types=[
        pltpu.VMEM_SHARED((N, D), jnp.float32),
        pltpu.VMEM((NBI, K), jnp.int32),
        pltpu.VMEM((NBI, K), jnp.int32),
        [pltpu.VMEM((K, D), jnp.float32)] * NBUF,
        [pltpu.SemaphoreType.DMA] * NBUF,
        [pltpu.SemaphoreType.DMA] * NBUF,
        [pltpu.SemaphoreType.DMA] * NBI,
        [pltpu.SemaphoreType.DMA] * NBI,
    ],
)
def _segment_sum_sc(h_hbm, src_hbm, dst_hbm, zero_hbm, out_hbm,
                    acc_sh, src_v, dst_v, rows, rsem, wsem, ssem, dsem):
    c = lax.axis_index("c")
    s = lax.axis_index("s")
    wid = c * NS + s

    @pl.when(s < NS - 1)
    def _():
        pltpu.sync_copy(zero_hbm.at[pl.ds(0, STR)],
                        acc_sh.at[pl.ds(s * STR, STR)])

    @pl.when(s == NS - 1)
    def _():
        pltpu.sync_copy(zero_hbm,
                        acc_sh.at[pl.ds((NS - 1) * STR, STR_LAST)])

    def start_idx(i, si):
        off = (wid * NCH + i) * K
        pltpu.make_async_copy(src_hbm.at[pl.ds(off, K)], src_v.at[si],
                              ssem[si]).start()
        pltpu.make_async_copy(dst_hbm.at[pl.ds(off, K)], dst_v.at[si],
                              dsem[si]).start()

    def wait_src(i, si):
        off = (wid * NCH + i) * K
        pltpu.make_async_copy(src_hbm.at[pl.ds(off, K)], src_v.at[si],
                              ssem[si]).wait()

    def wait_dst(i, si):
        off = (wid * NCH + i) * K
        pltpu.make_async_copy(dst_hbm.at[pl.ds(off, K)], dst_v.at[si],
                              dsem[si]).wait()

    def start_g(si, t):
        pltpu.make_async_copy(h_hbm.at[src_v.at[si]], rows[t],
                              rsem[t]).start()

    def wait_g(si, t):
        pltpu.make_async_copy(h_hbm.at[src_v.at[si]], rows[t],
                              rsem[t]).wait()

    def start_sc(si, t):
        pltpu.make_async_copy(rows[t], acc_sh.at[dst_v.at[si]],
                              wsem[t]).start(add=True)

    def wait_sc(si, t):
        pltpu.make_async_copy(rows[t], acc_sh.at[dst_v.at[si]],
                              wsem[t]).wait()

    for si in range(NBI):
        start_idx(si, si)
    for t in range(2):
        wait_src(t, t)
        start_g(t, t)
    plsc.subcore_barrier()

    NMAIN = (NCH // NBI) * NBI

    def chunk(j, carry):
        i0 = j * NBI
        for t in range(NBI):
            i = i0 + t
            rt = t % NBUF
            wait_g(t, rt)
            wait_dst(i, t)
            start_sc(t, rt)

            @pl.when(i + NBI < NCH)
            def _(i=i, t=t):
                start_idx(i + NBI, t)

            @pl.when(i >= 1)
            def _(t=t):
                wait_sc((t - 1) % NBI, (t - 1) % NBUF)

            @pl.when(i + 2 < NCH)
            def _(i=i, t=t):
                wait_src(i + 2, (t + 2) % NBI)
                start_g((t + 2) % NBI, (t + 2) % NBUF)

        return carry

    lax.fori_loop(0, NMAIN // NBI, chunk, 0)
    for i in range(NMAIN, NCH):
        t = i % NBI
        rt = i % NBUF
        wait_g(t, rt)
        wait_dst(i, t)
        start_sc(t, rt)
        wait_sc((t - 1) % NBI, (i - 1) % NBUF)
        if i + 2 < NCH:
            wait_src(i + 2, (i + 2) % NBI)
            start_g((i + 2) % NBI, (i + 2) % NBUF)
    wait_sc((NCH - 1) % NBI, (NCH - 1) % NBUF)
    plsc.subcore_barrier()

    @pl.when(s < NS - 1)
    def _():
        pltpu.sync_copy(acc_sh.at[pl.ds(s * STR, STR)],
                        out_hbm.at[c, pl.ds(s * STR, STR)])

    @pl.when(s == NS - 1)
    def _():
        pltpu.sync_copy(acc_sh.at[pl.ds((NS - 1) * STR, STR_LAST)],
                        out_hbm.at[c, pl.ds((NS - 1) * STR, STR_LAST)])


def _init_mm(x_ref, w_ref, o_ref):
    o_ref[...] = jnp.dot(x_ref[...], w_ref[...],
                         preferred_element_type=jnp.float32)


def _layer_tc(p_ref, h_ref, wg_ref, bg_ref, wr_ref, br_ref, g_ref, be_ref,
              o_ref):
    agg = p_ref[0] + p_ref[1]
    t = jnp.dot(agg, wg_ref[...], preferred_element_type=jnp.float32)
    new = jnp.maximum(t + bg_ref[...], 0.0)
    r = jnp.dot(h_ref[...], wr_ref[...], preferred_element_type=jnp.float32)
    new = new + jnp.maximum(r + br_ref[...], 0.0)
    mu = jnp.mean(new, axis=0, keepdims=True)
    cen = new - mu
    var = jnp.mean(cen * cen, axis=0, keepdims=True)
    o_ref[...] = g_ref[...] * cen * lax.rsqrt(var + EPS) + be_ref[...]


def kernel(x, edge_index, batch_size, W_init, Wg0, bg0, Wr0, br0, g0, be0,
           Wg1, bg1, Wr1, br1, g1, be1):
    src = edge_index[0]
    dst = edge_index[1]
    zeros = jnp.zeros((STR_LAST, D), jnp.float32)

    h = pl.pallas_call(
        _init_mm,
        out_shape=jax.ShapeDtypeStruct((N, D), jnp.float32),
    )(x, W_init)

    def layer(hh, Wg, bg, Wr, br, g, be):
        p = _segment_sum_sc(hh, src, dst, zeros)
        return pl.pallas_call(
            _layer_tc,
            out_shape=jax.ShapeDtypeStruct((N, D), jnp.float32),
        )(p, hh, Wg, bg.reshape(1, D), Wr, br.reshape(1, D),
          g.reshape(1, D), be.reshape(1, D))

    h = layer(h, Wg0, bg0, Wr0, br0, g0, be0)
    h = layer(h, Wg1, bg1, Wr1, br1, g1, be1)
    return h.reshape(BATCH, -1, D)


# speedup vs baseline: 1.0087x; 1.0087x over previous
import functools

import jax
import jax.numpy as jnp
from jax import lax
from jax.experimental import pallas as pl
from jax.experimental.pallas import tpu as pltpu
from jax.experimental.pallas import tpu_sc as plsc

N = 10000
D = 128
E = 320000
EPS = 1e-5
BATCH = 100

NC = 2
NS = 16
NW = NC * NS
EW = E // NW
K = 80
NCH = EW // K
NBUF = 3
NBI = 6
STR = 624
STR_LAST = N - (NS - 1) * STR

_mesh = plsc.VectorSubcoreMesh(core_axis_name="c", subcore_axis_name="s")


@functools.partial(
    pl.kernel,
    out_type=jax.ShapeDtypeStruct((NC, N, D), jnp.float32),
    mesh=_mesh,
    scratch_types=[
        pltpu.VMEM_SHARED((N, D), jnp.float32),
        pltpu.VMEM((NBI, K), jnp.int32),
        pltpu.VMEM((NBI, K), jnp.int32),
        [pltpu.VMEM((K, D), jnp.float32)] * NBUF,
        [pltpu.SemaphoreType.DMA] * NBUF,
        [pltpu.SemaphoreType.DMA] * NBI,
        [pltpu.SemaphoreType.DMA] * NBI,
    ],
)
def _segment_sum_sc(h_hbm, src_hbm, dst_hbm, zero_hbm, out_hbm,
                    acc_sh, src_v, dst_v, rows, rsem, ssem, dsem):
    c = lax.axis_index("c")
    s = lax.axis_index("s")
    wid = c * NS + s

    @pl.when(s < NS - 1)
    def _():
        pltpu.sync_copy(zero_hbm.at[pl.ds(0, STR)],
                        acc_sh.at[pl.ds(s * STR, STR)])

    @pl.when(s == NS - 1)
    def _():
        pltpu.sync_copy(zero_hbm,
                        acc_sh.at[pl.ds((NS - 1) * STR, STR_LAST)])

    def start_idx(i, si):
        off = (wid * NCH + i) * K
        pltpu.make_async_copy(src_hbm.at[pl.ds(off, K)], src_v.at[si],
                              ssem[si]).start()
        pltpu.make_async_copy(dst_hbm.at[pl.ds(off, K)], dst_v.at[si],
                              dsem[si]).start()

    def wait_src(i, si):
        off = (wid * NCH + i) * K
        pltpu.make_async_copy(src_hbm.at[pl.ds(off, K)], src_v.at[si],
                              ssem[si]).wait()

    def wait_dst(i, si):
        off = (wid * NCH + i) * K
        pltpu.make_async_copy(dst_hbm.at[pl.ds(off, K)], dst_v.at[si],
                              dsem[si]).wait()

    def start_g(si, t):
        pltpu.make_async_copy(h_hbm.at[src_v.at[si]], rows[t],
                              rsem[t]).start()

    def wait_g(si, t):
        pltpu.make_async_copy(h_hbm.at[src_v.at[si]], rows[t],
                              rsem[t]).wait()

    for si in range(NBI):
        start_idx(si, si)
    for t in range(NBUF):
        wait_src(t, t)
        start_g(t, t)
    plsc.subcore_barrier()

    NMAIN = (NCH // NBI) * NBI

    def chunk(j, carry):
        i0 = j * NBI
        for t in range(NBI):
            i = i0 + t
            rt = t % NBUF
            wait_g(t, rt)
            wait_dst(i, t)
            pltpu.sync_copy(rows[rt], acc_sh.at[dst_v.at[t]], add=True)

            @pl.when(i + NBI < NCH)
            def _(i=i, t=t):
                start_idx(i + NBI, t)

            @pl.when(i + NBUF < NCH)
            def _(i=i, t=t, rt=rt):
                wait_src(i + NBUF, (t + NBUF) % NBI)
                start_g((t + NBUF) % NBI, rt)

        return carry

    lax.fori_loop(0, NMAIN // NBI, chunk, 0)
    for i in range(NMAIN, NCH):
        t = i % NBI
        rt = i % NBUF
        wait_g(t, rt)
        wait_dst(i, t)
        pltpu.sync_copy(rows[rt], acc_sh.at[dst_v.at[t]], add=True)
        if i + NBUF < NCH:
            wait_src(i + NBUF, (i + NBUF) % NBI)
            start_g((i + NBUF) % NBI, rt)
    plsc.subcore_barrier()

    @pl.when(s < NS - 1)
    def _():
        pltpu.sync_copy(acc_sh.at[pl.ds(s * STR, STR)],
                        out_hbm.at[c, pl.ds(s * STR, STR)])

    @pl.when(s == NS - 1)
    def _():
        pltpu.sync_copy(acc_sh.at[pl.ds((NS - 1) * STR, STR_LAST)],
                        out_hbm.at[c, pl.ds((NS - 1) * STR, STR_LAST)])


def _init_mm(x_ref, w_ref, o_ref):
    o_ref[...] = jnp.dot(x_ref[...], w_ref[...],
                         preferred_element_type=jnp.float32)


def _layer_tc(p_ref, h_ref, wg_ref, bg_ref, wr_ref, br_ref, g_ref, be_ref,
              o_ref):
    agg = p_ref[0] + p_ref[1]
    t = jnp.dot(agg, wg_ref[...], preferred_element_type=jnp.float32)
    new = jnp.maximum(t + bg_ref[...], 0.0)
    r = jnp.dot(h_ref[...], wr_ref[...], preferred_element_type=jnp.float32)
    new = new + jnp.maximum(r + br_ref[...], 0.0)
    mu = jnp.mean(new, axis=0, keepdims=True)
    cen = new - mu
    var = jnp.mean(cen * cen, axis=0, keepdims=True)
    o_ref[...] = g_ref[...] * cen * lax.rsqrt(var + EPS) + be_ref[...]


def kernel(x, edge_index, batch_size, W_init, Wg0, bg0, Wr0, br0, g0, be0,
           Wg1, bg1, Wr1, br1, g1, be1):
    src = edge_index[0]
    dst = edge_index[1]
    zeros = jnp.zeros((STR_LAST, D), jnp.float32)

    h = pl.pallas_call(
        _init_mm,
        grid=(10,),
        in_specs=[pl.BlockSpec((N // 10, D), lambda i: (i, 0)),
                  pl.BlockSpec((D, D), lambda i: (0, 0))],
        out_specs=pl.BlockSpec((N // 10, D), lambda i: (i, 0)),
        out_shape=jax.ShapeDtypeStruct((N, D), jnp.float32),
    )(x, W_init)

    def layer(hh, Wg, bg, Wr, br, g, be):
        p = _segment_sum_sc(hh, src, dst, zeros)
        return pl.pallas_call(
            _layer_tc,
            out_shape=jax.ShapeDtypeStruct((N, D), jnp.float32),
        )(p, hh, Wg, bg.reshape(1, D), Wr, br.reshape(1, D),
          g.reshape(1, D), be.reshape(1, D))

    h = layer(h, Wg0, bg0, Wr0, br0, g0, be0)
    h = layer(h, Wg1, bg1, Wr1, br1, g1, be1)
    return h.reshape(BATCH, -1, D)


# speedup vs baseline: 1.0315x; 1.0226x over previous
import functools

import jax
import jax.numpy as jnp
from jax import lax
from jax.experimental import pallas as pl
from jax.experimental.pallas import tpu as pltpu
from jax.experimental.pallas import tpu_sc as plsc

N = 10000
D = 128
E = 320000
EPS = 1e-5
BATCH = 100

NC = 2
NS = 16
NW = NC * NS
EW = E // NW
K = 80
NCH = EW // K
NBUF = 3
NBI = 6
STR = 624
STR_LAST = N - (NS - 1) * STR

_mesh = plsc.VectorSubcoreMesh(core_axis_name="c", subcore_axis_name="s")


@functools.partial(
    pl.kernel,
    out_type=jax.ShapeDtypeStruct((NC, N, D), jnp.float32),
    mesh=_mesh,
    scratch_types=[
        pltpu.VMEM_SHARED((N, D), jnp.float32),
        pltpu.VMEM((NBI, K), jnp.int32),
        pltpu.VMEM((NBI, K), jnp.int32),
        [pltpu.VMEM((K, D), jnp.float32)] * NBUF,
        [pltpu.SemaphoreType.DMA] * NBUF,
        [pltpu.SemaphoreType.DMA] * NBI,
        [pltpu.SemaphoreType.DMA] * NBI,
    ],
)
def _segment_sum_sc(h_hbm, src_hbm, dst_hbm, zero_hbm, out_hbm,
                    acc_sh, src_v, dst_v, rows, rsem, ssem, dsem):
    c = lax.axis_index("c")
    s = lax.axis_index("s")
    wid = c * NS + s

    def start_idx(i, si):
        off = (wid * NCH + i) * K
        pltpu.make_async_copy(src_hbm.at[pl.ds(off, K)], src_v.at[si],
                              ssem[si]).start()
        pltpu.make_async_copy(dst_hbm.at[pl.ds(off, K)], dst_v.at[si],
                              dsem[si]).start()

    def wait_src(i, si):
        off = (wid * NCH + i) * K
        pltpu.make_async_copy(src_hbm.at[pl.ds(off, K)], src_v.at[si],
                              ssem[si]).wait()

    def wait_dst(i, si):
        off = (wid * NCH + i) * K
        pltpu.make_async_copy(dst_hbm.at[pl.ds(off, K)], dst_v.at[si],
                              dsem[si]).wait()

    def start_g(si, t):
        pltpu.make_async_copy(h_hbm.at[src_v.at[si]], rows[t],
                              rsem[t]).start()

    def wait_g(si, t):
        pltpu.make_async_copy(h_hbm.at[src_v.at[si]], rows[t],
                              rsem[t]).wait()

    for si in range(NBI):
        start_idx(si, si)
    for t in range(NBUF):
        wait_src(t, t)
        start_g(t, t)

    @pl.when(s < NS - 1)
    def _():
        pltpu.sync_copy(zero_hbm.at[pl.ds(0, STR)],
                        acc_sh.at[pl.ds(s * STR, STR)])

    @pl.when(s == NS - 1)
    def _():
        pltpu.sync_copy(zero_hbm,
                        acc_sh.at[pl.ds((NS - 1) * STR, STR_LAST)])

    plsc.subcore_barrier()

    NMAIN = (NCH // NBI) * NBI

    def chunk(j, carry):
        i0 = j * NBI
        for t in range(NBI):
            i = i0 + t
            rt = t % NBUF
            wait_g(t, rt)
            wait_dst(i, t)
            pltpu.sync_copy(rows[rt], acc_sh.at[dst_v.at[t]], add=True)

            @pl.when(i + NBI < NCH)
            def _(i=i, t=t):
                start_idx(i + NBI, t)

            @pl.when(i + NBUF < NCH)
            def _(i=i, t=t, rt=rt):
                wait_src(i + NBUF, (t + NBUF) % NBI)
                start_g((t + NBUF) % NBI, rt)

        return carry

    lax.fori_loop(0, NMAIN // NBI, chunk, 0)
    for i in range(NMAIN, NCH):
        t = i % NBI
        rt = i % NBUF
        wait_g(t, rt)
        wait_dst(i, t)
        pltpu.sync_copy(rows[rt], acc_sh.at[dst_v.at[t]], add=True)
        if i + NBUF < NCH:
            wait_src(i + NBUF, (i + NBUF) % NBI)
            start_g((i + NBUF) % NBI, rt)
    plsc.subcore_barrier()

    @pl.when(s < NS - 1)
    def _():
        pltpu.sync_copy(acc_sh.at[pl.ds(s * STR, STR)],
                        out_hbm.at[c, pl.ds(s * STR, STR)])

    @pl.when(s == NS - 1)
    def _():
        pltpu.sync_copy(acc_sh.at[pl.ds((NS - 1) * STR, STR_LAST)],
                        out_hbm.at[c, pl.ds((NS - 1) * STR, STR_LAST)])


def _init_mm(x_ref, w_ref, o_ref):
    o_ref[...] = jnp.dot(x_ref[...], w_ref[...],
                         preferred_element_type=jnp.float32)


def _layer_tc(p_ref, h_ref, wg_ref, bg_ref, wr_ref, br_ref, g_ref, be_ref,
              o_ref):
    agg = p_ref[0] + p_ref[1]
    t = jnp.dot(agg, wg_ref[...], preferred_element_type=jnp.float32)
    new = jnp.maximum(t + bg_ref[...], 0.0)
    r = jnp.dot(h_ref[...], wr_ref[...], preferred_element_type=jnp.float32)
    new = new + jnp.maximum(r + br_ref[...], 0.0)
    mu = jnp.mean(new, axis=0, keepdims=True)
    cen = new - mu
    var = jnp.mean(cen * cen, axis=0, keepdims=True)
    o_ref[...] = g_ref[...] * cen * lax.rsqrt(var + EPS) + be_ref[...]


def kernel(x, edge_index, batch_size, W_init, Wg0, bg0, Wr0, br0, g0, be0,
           Wg1, bg1, Wr1, br1, g1, be1):
    src = edge_index[0]
    dst = edge_index[1]
    zeros = jnp.zeros((STR_LAST, D), jnp.float32)

    h = pl.pallas_call(
        _init_mm,
        out_shape=jax.ShapeDtypeStruct((N, D), jnp.float32),
    )(x, W_init)

    def layer(hh, Wg, bg, Wr, br, g, be):
        p = _segment_sum_sc(hh, src, dst, zeros)
        return pl.pallas_call(
            _layer_tc,
            out_shape=jax.ShapeDtypeStruct((N, D), jnp.float32),
        )(p, hh, Wg, bg.reshape(1, D), Wr, br.reshape(1, D),
          g.reshape(1, D), be.reshape(1, D))

    h = layer(h, Wg0, bg0, Wr0, br0, g0, be0)
    h = layer(h, Wg1, bg1, Wr1, br1, g1, be1)
    return h.reshape(BATCH, -1, D)
